# baseline (speedup 1.0000x reference)
import functools

import jax
import jax.numpy as jnp
from jax import lax
from jax.experimental import pallas as pl
from jax.experimental.pallas import tpu as pltpu
from jax.experimental.pallas import tpu_sc as plsc

B, P, K = 2, 10000, 16
C = 128
COUT = 128
H = W = 100
NC, NS, L = 2, 16, 16
NW = NC * NS
CP = 8
CPK = CP * K
NCHUNK = P // CP
TPWP = 40
NBUF = 4
EIROWS = NW * TPWP


def _tc_matmul(x_flat, W_conv, b_row):
    BLK = 2000

    def body(x_ref, w_ref, b_ref, a_ref, n_ref):
        w = w_ref[...]
        w1 = w[:, :C]
        w2 = w[:, C:]
        xb = x_ref[...]
        dn = (((1,), (1,)), ((), ()))
        a_ref[...] = lax.dot_general(
            xb, w1 - w2, dn, preferred_element_type=jnp.float32) + b_ref[...]
        n_ref[...] = lax.dot_general(
            xb, w2, dn, preferred_element_type=jnp.float32)

    return pl.pallas_call(
        body,
        grid=(P // BLK,),
        in_specs=[
            pl.BlockSpec((BLK, C), lambda i: (i, 0)),
            pl.BlockSpec((COUT, 2 * C), lambda i: (0, 0)),
            pl.BlockSpec((1, COUT), lambda i: (0, 0)),
        ],
        out_specs=[
            pl.BlockSpec((BLK, COUT), lambda i: (i, 0)),
            pl.BlockSpec((BLK, COUT), lambda i: (i, 0)),
        ],
        out_shape=[
            jax.ShapeDtypeStruct((P, COUT), jnp.float32),
            jax.ShapeDtypeStruct((P, COUT), jnp.float32),
        ],
    )(x_flat, W_conv, b_row)


def _sc_gather_max(nv, ei2d):
    mesh = plsc.VectorSubcoreMesh(core_axis_name="c", subcore_axis_name="s")
    scratch = (
        [pltpu.VMEM((TPWP, CPK), jnp.int32)]
        + [pltpu.VMEM((CPK, COUT), jnp.float32) for _ in range(NBUF)]
        + [pltpu.VMEM((CP, COUT), jnp.float32) for _ in range(NBUF)]
        + [pltpu.SemaphoreType.DMA for _ in range(2 * NBUF)]
    )

    @functools.partial(
        pl.kernel,
        out_type=jax.ShapeDtypeStruct((P, COUT), jnp.float32),
        mesh=mesh,
        scratch_types=scratch,
    )
    def k(nv_hbm, ei_hbm, out_hbm, idxs, *bufs):
        g = bufs[0:NBUF]
        o = bufs[NBUF:2 * NBUF]
        sg = bufs[2 * NBUF:3 * NBUF]
        so = bufs[3 * NBUF:4 * NBUF]
        wid = lax.axis_index("s") * NC + lax.axis_index("c")
        lo = wid * TPWP
        nw = jnp.clip(NCHUNK - lo, 0, TPWP)

        pltpu.sync_copy(ei_hbm.at[pl.ds(lo, TPWP)], idxs)

        def fire(b, c):
            pltpu.async_copy(nv_hbm.at[idxs.at[c]], g[b], sg[b])

        for b in range(NBUF):
            @pl.when(b < nw)
            def _(b=b):
                fire(b, b)

        def outer(t, carry):
            base = t * NBUF
            for b in range(NBUF):
                c = base + b

                @pl.when(c < nw)
                def _(b=b, c=c):
                    pltpu.make_async_copy(nv_hbm.at[idxs.at[c]], g[b], sg[b]).wait()

                    @pl.when(c >= NBUF)
                    def _():
                        pltpu.make_async_copy(
                            o[b], out_hbm.at[pl.ds(0, CP)], so[b]).wait()

                    def point_body(p, c2):
                        rbase = p * K
                        for cb in range(COUT // L):
                            sl = pl.ds(cb * L, L)
                            m = g[b][rbase, sl]
                            for j in range(1, K):
                                m = jnp.maximum(m, g[b][rbase + j, sl])
                            o[b][p, sl] = m
                        return c2

                    lax.fori_loop(0, CP, point_body, 0)
                    pltpu.async_copy(
                        o[b], out_hbm.at[pl.ds((lo + c) * CP, CP)], so[b])

                    @pl.when(c + NBUF < nw)
                    def _(b=b, c=c):
                        fire(b, c + NBUF)

            return carry

        lax.fori_loop(0, TPWP // NBUF, outer, 0)

        for b in range(NBUF):
            pltpu.make_async_copy(o[b], out_hbm.at[pl.ds(0, CP)], so[b]).wait()

    return k(nv, ei2d)


def _tc_finish(m, a):
    BLK = 2000

    def body(m_ref, a_ref, o_ref):
        o_ref[...] = jnp.maximum(m_ref[...] + a_ref[...], 0.0)

    return pl.pallas_call(
        body,
        grid=(P // BLK,),
        in_specs=[
            pl.BlockSpec((BLK, COUT), lambda i: (i, 0)),
            pl.BlockSpec((BLK, COUT), lambda i: (i, 0)),
        ],
        out_specs=pl.BlockSpec((BLK, COUT), lambda i: (i, 0)),
        out_shape=jax.ShapeDtypeStruct((P, COUT), jnp.float32),
    )(m, a)


def kernel(x, edge_index, size, W_conv, b_conv):
    del size
    b_row = b_conv.reshape(1, COUT)
    pad = jnp.zeros((EIROWS * CPK - P * K,), dtype=edge_index.dtype)
    mats = [_tc_matmul(x[b], W_conv, b_row) for b in range(B)]
    eis = [jnp.concatenate([edge_index[b].reshape(P * K), pad])
           .reshape(EIROWS, CPK) for b in range(B)]
    ms = [_sc_gather_max(mats[b][1], eis[b]) for b in range(B)]
    ys = [_tc_finish(ms[b], mats[b][0]) for b in range(B)]
    out = jnp.stack([y.T.reshape(COUT, H, W) for y in ys])
    return out


# speedup vs baseline: 1.0936x; 1.0936x over previous
import functools

import jax
import jax.numpy as jnp
from jax import lax
from jax.experimental import pallas as pl
from jax.experimental.pallas import tpu as pltpu
from jax.experimental.pallas import tpu_sc as plsc

B, P, K = 2, 10000, 16
C = 128
COUT = 128
H = W = 100
BP = B * P
NC, NS, L = 2, 16, 16
NW = NC * NS
CP = 8
CPK = CP * K
NCHUNK = BP // CP
BCHUNK = P // CP


def _tc_matmul(x_flat, W_conv, b_row):
    BLK = 2000

    def body(x_ref, w_ref, b_ref, a_ref, n_ref):
        w = w_ref[...]
        w1 = w[:, :C]
        w2 = w[:, C:]
        xb = x_ref[...]
        dn = (((1,), (1,)), ((), ()))
        a_ref[...] = lax.dot_general(
            xb, w1 - w2, dn, preferred_element_type=jnp.float32) + b_ref[...]
        n_ref[...] = lax.dot_general(
            xb, w2, dn, preferred_element_type=jnp.float32)

    return pl.pallas_call(
        body,
        grid=(BP // BLK,),
        in_specs=[
            pl.BlockSpec((BLK, C), lambda i: (i, 0)),
            pl.BlockSpec((COUT, 2 * C), lambda i: (0, 0)),
            pl.BlockSpec((1, COUT), lambda i: (0, 0)),
        ],
        out_specs=[
            pl.BlockSpec((BLK, COUT), lambda i: (i, 0)),
            pl.BlockSpec((BLK, COUT), lambda i: (i, 0)),
        ],
        out_shape=[
            jax.ShapeDtypeStruct((BP, COUT), jnp.float32),
            jax.ShapeDtypeStruct((BP, COUT), jnp.float32),
        ],
    )(x_flat, W_conv, b_row)


def _sc_gather_max(nv, ei2d):
    mesh = plsc.VectorSubcoreMesh(core_axis_name="c", subcore_axis_name="s")
    scratch = (
        [pltpu.VMEM((CP, K), jnp.int32) for _ in range(2)]
        + [pltpu.VMEM((CPK, COUT), jnp.float32) for _ in range(2)]
        + [pltpu.VMEM((CP, COUT), jnp.float32) for _ in range(2)]
        + [pltpu.SemaphoreType.DMA for _ in range(6)]
    )

    @functools.partial(
        pl.kernel,
        out_type=jax.ShapeDtypeStruct((BP, COUT), jnp.float32),
        mesh=mesh,
        scratch_types=scratch,
    )
    def k(nv_hbm, ei_hbm, out_hbm, *bufs):
        I = bufs[0:2]
        G = bufs[2:4]
        O = bufs[4:6]
        si = bufs[6:8]
        sg = bufs[8:10]
        so = bufs[10:12]
        wid = lax.axis_index("s") * NC + lax.axis_index("c")
        lo = (NCHUNK * wid) // NW
        hi = (NCHUNK * (wid + 1)) // NW
        nw = hi - lo
        end = lo + nw

        def fire_idx(s, c):
            pltpu.async_copy(ei_hbm.at[pl.ds(c * CP, CP)], I[s], si[s])

        def wait_idx(s):
            pltpu.make_async_copy(ei_hbm.at[pl.ds(0, CP)], I[s], si[s]).wait()

        def fire_gathers(s, c):
            off = jnp.where(c >= BCHUNK, P, 0)

            def gp(p, carry):
                idxv = I[s][p, :] + off
                pltpu.async_copy(
                    nv_hbm.at[idxv], G[s].at[pl.ds(p * K, K)], sg[s])
                return carry

            lax.fori_loop(0, CP, gp, 0)

        def wait_gathers(s):
            pltpu.make_async_copy(nv_hbm.at[pl.ds(0, CPK)], G[s], sg[s]).wait()

        def fire_out(s, c):
            pltpu.async_copy(O[s], out_hbm.at[pl.ds(c * CP, CP)], so[s])

        def wait_out(s):
            pltpu.make_async_copy(O[s], out_hbm.at[pl.ds(0, CP)], so[s]).wait()

        def compute(s):
            def point_body(p, carry):
                rbase = p * K
                for cb in range(COUT // L):
                    sl = pl.ds(cb * L, L)
                    m = G[s][rbase, sl]
                    for j in range(1, K):
                        m = jnp.maximum(m, G[s][rbase + j, sl])
                    O[s][p, sl] = m
                return carry

            lax.fori_loop(0, CP, point_body, 0)

        def step(s, c):
            @pl.when(c + 1 < end)
            def _():
                wait_idx(1 - s)
                fire_gathers(1 - s, c + 1)

            @pl.when(c + 2 < end)
            def _():
                fire_idx(s, c + 2)

            wait_gathers(s)

            @pl.when(c - 2 >= lo)
            def _():
                wait_out(s)

            compute(s)
            fire_out(s, c)

        fire_idx(0, lo)
        fire_idx(1, lo + 1)
        wait_idx(0)
        fire_gathers(0, lo)

        def pair_body(t2, carry):
            c = lo + 2 * t2
            step(0, c)
            step(1, c + 1)
            return carry

        lax.fori_loop(0, nw // 2, pair_body, 0)

        @pl.when(nw % 2 == 1)
        def _():
            step(0, end - 1)

        wait_out(0)
        wait_out(1)

    return k(nv, ei2d)


def _tc_finish(m, a):
    BLK = 2000

    def body(m_ref, a_ref, o_ref):
        o_ref[...] = jnp.maximum(m_ref[...] + a_ref[...], 0.0)

    return pl.pallas_call(
        body,
        grid=(BP // BLK,),
        in_specs=[
            pl.BlockSpec((BLK, COUT), lambda i: (i, 0)),
            pl.BlockSpec((BLK, COUT), lambda i: (i, 0)),
        ],
        out_specs=pl.BlockSpec((BLK, COUT), lambda i: (i, 0)),
        out_shape=jax.ShapeDtypeStruct((BP, COUT), jnp.float32),
    )(m, a)


def kernel(x, edge_index, size, W_conv, b_conv):
    del size
    x_flat = x.reshape(BP, C)
    ei2d = edge_index.reshape(BP, K)
    a_mat, nv = _tc_matmul(x_flat, W_conv, b_conv.reshape(1, COUT))
    m = _sc_gather_max(nv, ei2d)
    y3 = _tc_finish(m, a_mat).reshape(B, P, COUT)
    return jnp.transpose(y3, (0, 2, 1)).reshape(B, COUT, H, W)


# speedup vs baseline: 1.1483x; 1.0500x over previous
import functools

import jax
import jax.numpy as jnp
from jax import lax
from jax.experimental import pallas as pl
from jax.experimental.pallas import tpu as pltpu
from jax.experimental.pallas import tpu_sc as plsc

B, P, K = 2, 10000, 16
C = 128
COUT = 128
H = W = 100
BP = B * P
NC, NS, L = 2, 16, 16
NW = NC * NS
CP = 8
CPK = CP * K
NCHUNK = BP // CP
BCHUNK = P // CP


def _tc_matmul(x_flat, W_conv, b_row):
    BLK = 2000

    def body(x_ref, w_ref, b_ref, a_ref, n_ref):
        w = w_ref[...]
        w1 = w[:, :C]
        w2 = w[:, C:]
        xb = x_ref[...]
        dn = (((1,), (1,)), ((), ()))
        a_ref[...] = lax.dot_general(
            xb, w1 - w2, dn, preferred_element_type=jnp.float32) + b_ref[...]
        n_ref[...] = lax.dot_general(
            xb, w2, dn, preferred_element_type=jnp.float32)

    return pl.pallas_call(
        body,
        grid=(BP // BLK,),
        in_specs=[
            pl.BlockSpec((BLK, C), lambda i: (i, 0)),
            pl.BlockSpec((COUT, 2 * C), lambda i: (0, 0)),
            pl.BlockSpec((1, COUT), lambda i: (0, 0)),
        ],
        out_specs=[
            pl.BlockSpec((BLK, COUT), lambda i: (i, 0)),
            pl.BlockSpec((BLK, COUT), lambda i: (i, 0)),
        ],
        out_shape=[
            jax.ShapeDtypeStruct((BP, COUT), jnp.float32),
            jax.ShapeDtypeStruct((BP, COUT), jnp.float32),
        ],
    )(x_flat, W_conv, b_row)


def _sc_gather_max(nv, ei2d):
    mesh = plsc.VectorSubcoreMesh(core_axis_name="c", subcore_axis_name="s")
    NBUF = 4
    scratch = (
        [pltpu.VMEM((CP, K), jnp.int32) for _ in range(NBUF)]
        + [pltpu.VMEM((CPK,), jnp.int32) for _ in range(NBUF)]
        + [pltpu.VMEM((CPK, COUT), jnp.float32) for _ in range(NBUF)]
        + [pltpu.VMEM((CP, COUT), jnp.float32) for _ in range(NBUF)]
        + [pltpu.SemaphoreType.DMA for _ in range(3 * NBUF)]
    )

    @functools.partial(
        pl.kernel,
        out_type=jax.ShapeDtypeStruct((BP, COUT), jnp.float32),
        mesh=mesh,
        scratch_types=scratch,
    )
    def k(nv_hbm, ei_hbm, out_hbm, *bufs):
        I = bufs[0:NBUF]
        F = bufs[NBUF:2 * NBUF]
        G = bufs[2 * NBUF:3 * NBUF]
        O = bufs[3 * NBUF:4 * NBUF]
        si = bufs[4 * NBUF:5 * NBUF]
        sg = bufs[5 * NBUF:6 * NBUF]
        so = bufs[6 * NBUF:7 * NBUF]
        wid = lax.axis_index("s") * NC + lax.axis_index("c")
        lo = (NCHUNK * wid) // NW
        nw = (NCHUNK * (wid + 1)) // NW - lo

        def fire_idx(b, c):
            pltpu.async_copy(ei_hbm.at[pl.ds((lo + c) * CP, CP)], I[b], si[b])

        def fire(b, c):
            pltpu.make_async_copy(
                ei_hbm.at[pl.ds(0, CP)], I[b], si[b]).wait()
            off = jnp.where(lo + c >= BCHUNK, P, 0)
            for p in range(CP):
                F[b][pl.ds(p * K, K)] = I[b][p, :] + off

            @pl.when(c + NBUF < nw)
            def _():
                fire_idx(b, c + NBUF)

            pltpu.async_copy(nv_hbm.at[F[b]], G[b], sg[b])

        for b in range(NBUF):
            fire_idx(b, b)
        for b in range(NBUF):
            fire(b, b)

        def outer(t, carry):
            base = t * NBUF
            for b in range(NBUF):
                c = base + b

                @pl.when(c < nw)
                def _(b=b, c=c):
                    pltpu.make_async_copy(nv_hbm.at[F[b]], G[b], sg[b]).wait()

                    @pl.when(c >= NBUF)
                    def _():
                        pltpu.make_async_copy(
                            O[b], out_hbm.at[pl.ds(0, CP)], so[b]).wait()

                    def point_body(p, carry2):
                        rbase = p * K
                        for cb in range(COUT // L):
                            sl = pl.ds(cb * L, L)
                            m = G[b][rbase, sl]
                            for j in range(1, K):
                                m = jnp.maximum(m, G[b][rbase + j, sl])
                            O[b][p, sl] = m
                        return carry2

                    lax.fori_loop(0, CP, point_body, 0)
                    pltpu.async_copy(
                        O[b], out_hbm.at[pl.ds((lo + c) * CP, CP)], so[b])

                    @pl.when(c + NBUF < nw)
                    def _(b=b, c=c):
                        fire(b, c + NBUF)

            return carry

        lax.fori_loop(0, (nw + NBUF - 1) // NBUF, outer, 0)

        for b in range(NBUF):
            pltpu.make_async_copy(O[b], out_hbm.at[pl.ds(0, CP)], so[b]).wait()

    return k(nv, ei2d)


def _tc_finish(m, a):
    BLK = 2000

    def body(m_ref, a_ref, o_ref):
        o_ref[...] = jnp.maximum(m_ref[...] + a_ref[...], 0.0)

    return pl.pallas_call(
        body,
        grid=(BP // BLK,),
        in_specs=[
            pl.BlockSpec((BLK, COUT), lambda i: (i, 0)),
            pl.BlockSpec((BLK, COUT), lambda i: (i, 0)),
        ],
        out_specs=pl.BlockSpec((BLK, COUT), lambda i: (i, 0)),
        out_shape=jax.ShapeDtypeStruct((BP, COUT), jnp.float32),
    )(m, a)


def kernel(x, edge_index, size, W_conv, b_conv):
    del size
    x_flat = x.reshape(BP, C)
    ei2d = edge_index.reshape(BP, K)
    a_mat, nv = _tc_matmul(x_flat, W_conv, b_conv.reshape(1, COUT))
    m = _sc_gather_max(nv, ei2d)
    y3 = _tc_finish(m, a_mat).reshape(B, P, COUT)
    return jnp.transpose(y3, (0, 2, 1)).reshape(B, COUT, H, W)
